# Initial kernel scaffold; baseline (speedup 1.0000x reference)
import jax
import jax.numpy as jnp
from jax.experimental import pallas as pl


def kernel(basis_vector_bank, task_f, img_f, params):
    raise NotImplementedError("write your pallas kernel here")


# speedup vs baseline: 11.8540x; 11.8540x over previous
import functools

import jax
import jax.numpy as jnp
from jax.experimental import pallas as pl
from jax.experimental.pallas import tpu as pltpu

_B, _INC, _H, _W = 2, 64, 32, 32
_ED, _FN, _SN, _VN = 256, 8, 4, 16
_OD = 16
_L = 20 * _ED
_HW = _H * _W

_F32 = jnp.float32


def _dotg(x, w):
    return jax.lax.dot_general(x, w, (((1,), (1,)), ((), ())),
                               preferred_element_type=_F32)


def _leaky(x):
    return jnp.where(x >= 0, x, 0.2 * x)


def _ln(x, g, b):
    m = jnp.mean(x, axis=-1, keepdims=True)
    v = jnp.mean((x - m) ** 2, axis=-1, keepdims=True)
    return (x - m) / jnp.sqrt(v + 1e-5) * g + b


def _conv_taps(x, wtaps, d):
    yy = jax.lax.broadcasted_iota(jnp.int32, (1, _HW), 1) // _W
    xx = jax.lax.broadcasted_iota(jnp.int32, (1, _HW), 1) % _W
    acc = None
    for ky in range(3):
        for kx in range(3):
            dy, dx = (ky - 1) * d, (kx - 1) * d
            off = dy * _W + dx
            xs = jnp.roll(x, -off, axis=1) if off else x
            valid = ((yy + dy >= 0) & (yy + dy < _H)
                     & (xx + dx >= 0) & (xx + dx < _W))
            xs = jnp.where(valid, xs, 0.0)
            t = jnp.dot(wtaps[3 * ky + kx], xs, preferred_element_type=_F32)
            acc = t if acc is None else acc + t
    return acc


def _colrow(col, w):
    return jax.lax.dot_general(col, w, (((0,), (1,)), ((), ())),
                               preferred_element_type=_F32)


def _task_encoder(xab, w1, w2taps):
    il = jax.lax.broadcasted_iota(jnp.int32, (1, _L), 1)
    xl = jnp.where(il > 0, jnp.roll(xab, 1, axis=1), 0.0)
    xr = jnp.where(il < _L - 1, jnp.roll(xab, -1, axis=1), 0.0)
    h1 = _leaky(w1[:, 0:1] * xl + w1[:, 1:2] * xab + w1[:, 2:3] * xr)
    s = jnp.sum(h1, axis=1, keepdims=True)
    e0 = h1[:, 0:1]
    el = h1[:, _L - 1:_L]
    out = (_colrow(s - el, w2taps[0]) + _colrow(s, w2taps[1])
           + _colrow(s - e0, w2taps[2]))
    return out / float(_L)


def _bn_act(x, g, b):
    m = jnp.mean(x, axis=(0, 2), keepdims=True)
    v = jnp.mean((x - m) ** 2, axis=(0, 2), keepdims=True)
    return _leaky((x - m) / jnp.sqrt(v + 1e-5) * g[None] + b[None])


def _prelude_kernel(task_ref, img_ref, bank_ref, gum_ref,
                    aptw1_ref, aptw2_ref, apiw_ref, apibng_ref, apibnb_ref,
                    atiw1_ref, atib1_ref, atilng_ref, atilnb_ref,
                    atiw2_ref, atib2_ref, protos_ref,
                    fptw1_ref, fptw2_ref,
                    fpiw1_ref, fpibn1g_ref, fpibn1b_ref,
                    fpiw2_ref, fpibn2g_ref, fpibn2b_ref,
                    fmlpw1_ref, fmlpb1_ref, fmlplng_ref, fmlplnb_ref,
                    fmlpw2_ref, fmlpb2_ref,
                    m1w1_ref, m1b1_ref, m1lng_ref, m1lnb_ref, m1w2_ref,
                    m3w1_ref, m3b1_ref, m3lng_ref, m3lnb_ref, m3w29_ref,
                    w1a_ref, w3a_ref, idxo_ref):
    xa = task_ref[:]
    img = img_ref[:]
    gum = gum_ref[:]

    aptw1 = aptw1_ref[:]
    aptw2 = aptw2_ref[:]
    tf1 = jnp.concatenate(
        [_task_encoder(xa[b:b + 1], aptw1, aptw2) for b in range(_B)],
        axis=0)

    apw = apiw_ref[:]
    hab = jnp.concatenate(
        [_conv_taps(img[b], [apw[k] for k in range(9)], 1)[None]
         for b in range(_B)], axis=0)
    hab = _bn_act(hab, apibng_ref[:], apibnb_ref[:])
    if1 = jnp.mean(hab, axis=2)

    hcat = jnp.concatenate([tf1, if1], axis=1)
    h = jax.nn.relu(_ln(_dotg(hcat, atiw1_ref[:]) + atib1_ref[:],
                        atilng_ref[:], atilnb_ref[:]))
    timg2 = _dotg(h, atiw2_ref[:]) + atib2_ref[:]

    protos = protos_ref[:]
    pr = jnp.sqrt(jnp.sum(protos * protos, axis=1, keepdims=True))
    pn = protos / jnp.maximum(pr, 1e-8)
    nflat = _OD * _FN
    gt_r = jax.lax.broadcasted_iota(jnp.int32, (_FN, nflat), 0)
    gt_c = jax.lax.broadcasted_iota(jnp.int32, (_FN, nflat), 1)
    gsel = jnp.where(gt_r == gt_c % _FN, 1.0, 0.0)
    ee_r = jax.lax.broadcasted_iota(jnp.int32, (_OD, nflat), 0)
    ee_c = jax.lax.broadcasted_iota(jnp.int32, (_OD, nflat), 1)
    eexp = jnp.where(ee_r == ee_c // _FN, 1.0, 0.0)
    pna = jnp.dot(pn, eexp, preferred_element_type=_F32)
    it4 = jax.lax.broadcasted_iota(jnp.int32, (_FN, _SN), 1)
    iu8 = jax.lax.broadcasted_iota(jnp.int32, (_FN, _FN), 0)
    jv8 = jax.lax.broadcasted_iota(jnp.int32, (_FN, _FN), 1)
    idx_b = []
    ranks_b = []
    for b in range(_B):
        t2 = timg2[b:b + 1]
        u = gsel * t2
        nt2 = jax.lax.dot_general(gsel, t2 * t2, (((1,), (1,)), ((), ())),
                                  preferred_element_type=_F32)
        denom = jnp.maximum(jnp.sqrt(nt2), 1e-8)
        st_ft = jax.lax.dot_general(u, pna, (((1,), (1,)), ((), ())),
                                    preferred_element_type=_F32)
        zb = st_ft / denom + gum[b]
        mx = jnp.max(zb, axis=1, keepdims=True)
        idxb = jnp.min(jnp.where(zb == mx, it4, _SN), axis=1,
                       keepdims=True)
        ohb = jnp.where(it4 == idxb, 1.0, 0.0)
        same = jax.lax.dot_general(ohb, ohb, (((1,), (1,)), ((), ())),
                                   preferred_element_type=_F32)
        ranksb = jnp.sum(jnp.where(jv8 < iu8, same, 0.0), axis=1,
                         keepdims=True)
        idx_b.append(idxb)
        ranks_b.append(ranksb)

    fptw1 = fptw1_ref[:]
    fptw2 = fptw2_ref[:]
    te_rows = []
    for b in range(_B):
        xab = xa[b:b + 1]
        rms = jnp.sqrt(jnp.mean(xab * xab, axis=(0, 1), keepdims=True))
        xtb = xab / (rms + 1e-8)
        te_rows.append(_task_encoder(xtb, fptw1, fptw2))
    te = jnp.concatenate(te_rows, axis=0)

    fp1 = fpiw1_ref[:]
    hc = jnp.concatenate(
        [_conv_taps(img[b], [fp1[k] for k in range(9)], 1)[None]
         for b in range(_B)], axis=0)
    hc = _bn_act(hc, fpibn1g_ref[:], fpibn1b_ref[:])
    fp2 = fpiw2_ref[:]
    h2 = jnp.concatenate(
        [_conv_taps(hc[b], [fp2[k] for k in range(9)], 1)[None]
         for b in range(_B)], axis=0)
    h2 = _bn_act(h2, fpibn2g_ref[:], fpibn2b_ref[:])
    ie = jnp.mean(h2, axis=2)

    fcat = jnp.concatenate([te, ie], axis=1)
    h = jax.nn.relu(_ln(_dotg(fcat, fmlpw1_ref[:]) + fmlpb1_ref[:],
                        fmlplng_ref[:], fmlplnb_ref[:]))
    ff = _dotg(h, fmlpw2_ref[:]) + fmlpb2_ref[:]

    bank = bank_ref[:]
    rn = jnp.sqrt(jnp.sum(bank * bank, axis=1, keepdims=True))
    nbank = bank / jnp.maximum(rn, 1e-12)
    iu = jax.lax.broadcasted_iota(jnp.int32, (_SN * _VN, _SN * _VN), 0)
    iv = jax.lax.broadcasted_iota(jnp.int32, (_SN * _VN, _SN * _VN), 1)
    same_type = (iu // _VN) == (iv // _VN)
    iv64 = jax.lax.broadcasted_iota(jnp.int32, (1, _SN * _VN), 1)
    tdiv = iv64 // _VN
    ident64 = jnp.where(iu == iv, 1.0, 0.0)
    ones64 = jnp.full((_SN * _VN, _SN * _VN), 1.0, _F32)
    bvf_rows = []
    for b in range(_B):
        qb = ff[b:b + 1]
        qn = qb / jnp.maximum(
            jnp.sqrt(jnp.sum(qb * qb, axis=1, keepdims=True)), 1e-12)
        callb = jax.lax.dot_general(qn, nbank, (((1,), (1,)), ((), ())),
                                    preferred_element_type=_F32)
        cu = jnp.dot(ident64 * callb, ones64, preferred_element_type=_F32)
        beats = (cu > callb) | ((cu == callb) & (iu < iv))
        rank64 = jnp.sum(jnp.where(beats & same_type, 1.0, 0.0), axis=0,
                         keepdims=True)
        selm = jnp.where((rank64 == ranks_b[b]) & (tdiv == idx_b[b]),
                         1.0, 0.0)
        bvf_rows.append(jnp.dot(selm, bank, preferred_element_type=_F32))
    bvf = jnp.concatenate(bvf_rows, axis=0)

    h1m = jax.nn.relu(_ln(_dotg(bvf, m1w1_ref[:]) + m1b1_ref[:],
                          m1lng_ref[:], m1lnb_ref[:]))
    w1a_ref[:] = _dotg(h1m, m1w2_ref[:])
    h3m = jax.nn.relu(_ln(_dotg(bvf, m3w1_ref[:]) + m3b1_ref[:],
                          m3lng_ref[:], m3lnb_ref[:]))
    for k in range(9):
        w3a_ref[k] = _dotg(h3m, m3w29_ref[k])

    idxo_ref[:] = jnp.concatenate(
        [idx_b[0], idx_b[1], jnp.zeros((_FN, 6), jnp.int32)], axis=1)


def _dynconv_kernel(idx_ref, img_ref, w1a_ref, w3a_ref,
                    blrg_ref, blrb_ref, zcw_ref, zcb_ref,
                    out_ref, dps_ref):
    img = img_ref[:]
    for i in range(_B * _FN):
        b, f = divmod(i, _FN)
        idxv = idx_ref[f, b]
        xb = img[b]
        w1m = w1a_ref[pl.ds(i * _INC, _INC), :]
        w3taps = [w3a_ref[k, pl.ds(i * _INC, _INC), :] for k in range(9)]

        def br0():
            return jnp.dot(w1m, xb, preferred_element_type=_F32)

        def brd(d):
            def go():
                return _conv_taps(xb, w3taps, d)
            return go

        dps_ref[i] = jax.lax.switch(idxv, (br0, brd(1), brd(2), brd(3)))

    dv = dps_ref[:]
    m = jnp.mean(dv, axis=(0, 2), keepdims=True)
    v = jnp.mean((dv - m) ** 2, axis=(0, 2), keepdims=True)
    xn = _leaky((dv - m) / jnp.sqrt(v + 1e-5) * blrg_ref[:][None]
                + blrb_ref[:][None])
    zcw = zcw_ref[:]
    zcb = zcb_ref[:]
    for b in range(_B):
        bd = jnp.mean(xn[b * _FN:(b + 1) * _FN], axis=0)
        out_ref[b] = (jnp.dot(zcw, bd, preferred_element_type=_F32)
                      + zcb + img[b])


@jax.jit
def kernel(basis_vector_bank, task_f, img_f, params):
    p = params
    task2d = task_f.reshape(_B, _L).astype(_F32)
    img2d = img_f.reshape(_B, _INC, _HW).astype(_F32)
    bankflat = basis_vector_bank.reshape(_SN * _VN, _ED).astype(_F32)
    gum = jax.random.gumbel(jax.random.key(42), (_B, _FN, _SN), dtype=_F32)

    prelude_ins = (
        task2d, img2d, bankflat, gum,
        p['a_pt_w1'].reshape(4, 3),
        p['a_pt_w2'].transpose(2, 0, 1),
        p['a_pi_w'].transpose(2, 3, 0, 1).reshape(9, _FN, _INC),
        p['a_pi_bn_g'].reshape(_FN, 1), p['a_pi_bn_b'].reshape(_FN, 1),
        p['ati_w1'], p['ati_b1'].reshape(1, -1),
        p['ati_ln_g'].reshape(1, -1), p['ati_ln_b'].reshape(1, -1),
        p['ati_w2'], p['ati_b2'].reshape(1, -1),
        p['protos'],
        p['f_pt_w1'].reshape(_INC, 3),
        p['f_pt_w2'].transpose(2, 0, 1),
        p['f_pi_w1'].transpose(2, 3, 0, 1).reshape(9, _INC, _INC),
        p['f_pi_bn1_g'].reshape(_INC, 1), p['f_pi_bn1_b'].reshape(_INC, 1),
        p['f_pi_w2'].transpose(2, 3, 0, 1).reshape(9, 2 * _INC, _INC),
        p['f_pi_bn2_g'].reshape(2 * _INC, 1), p['f_pi_bn2_b'].reshape(2 * _INC, 1),
        p['f_mlp_w1'], p['f_mlp_b1'].reshape(1, -1),
        p['f_mlp_ln_g'].reshape(1, -1), p['f_mlp_ln_b'].reshape(1, -1),
        p['f_mlp_w2'], p['f_mlp_b2'].reshape(1, -1),
        p['m1_w1'], p['m1_b1'].reshape(1, -1),
        p['m1_ln_g'].reshape(1, -1), p['m1_ln_b'].reshape(1, -1),
        p['m1_w2'],
        p['m3_w1'], p['m3_b1'].reshape(1, -1),
        p['m3_ln_g'].reshape(1, -1), p['m3_ln_b'].reshape(1, -1),
        p['m3_w2'].reshape(_INC * _INC, 9, _INC).transpose(1, 0, 2),
    )
    w1a, w3a, idx8 = pl.pallas_call(
        _prelude_kernel,
        out_shape=(
            jax.ShapeDtypeStruct((_B * _FN, _INC * _INC), _F32),
            jax.ShapeDtypeStruct((9, _B * _FN, _INC * _INC), _F32),
            jax.ShapeDtypeStruct((_FN, 8), jnp.int32),
        ),
    )(*prelude_ins)
    w1a = w1a.reshape(_B * _FN * _INC, _INC)
    w3a = w3a.reshape(9, _B * _FN * _INC, _INC)

    out2d = pl.pallas_call(
        _dynconv_kernel,
        in_specs=[
            pl.BlockSpec(memory_space=pltpu.SMEM),
            pl.BlockSpec(memory_space=pltpu.VMEM),
            pl.BlockSpec(memory_space=pltpu.VMEM),
            pl.BlockSpec(memory_space=pltpu.VMEM),
            pl.BlockSpec(memory_space=pltpu.VMEM),
            pl.BlockSpec(memory_space=pltpu.VMEM),
            pl.BlockSpec(memory_space=pltpu.VMEM),
            pl.BlockSpec(memory_space=pltpu.VMEM),
        ],
        out_shape=jax.ShapeDtypeStruct((_B, _INC, _HW), _F32),
        scratch_shapes=[pltpu.VMEM((_B * _FN, _INC, _HW), _F32)],
    )(idx8, img2d, w1a, w3a,
      p['blr_g'].reshape(_INC, 1), p['blr_b'].reshape(_INC, 1),
      p['zc_w'].reshape(_INC, _INC), p['zc_b'].reshape(_INC, 1))

    return out2d.reshape(_B, _INC, _H, _W)
